# Initial kernel scaffold; baseline (speedup 1.0000x reference)
import jax
import jax.numpy as jnp
from jax.experimental import pallas as pl


def kernel(x_cur, log_w, y, noise, u, A, C):
    raise NotImplementedError("write your pallas kernel here")


# speedup vs baseline: 3.4787x; 3.4787x over previous
import functools
import math

import jax
import jax.numpy as jnp
import numpy as np
from jax.experimental import pallas as pl
from jax.experimental.pallas import tpu as pltpu

B, N, D, D_OBS = 16, 100000, 8, 4
BN = 12500
NB = N // BN

_LOGP_CONST = np.float32(-D_OBS * np.log(np.float32(0.5))
                         - 0.5 * D_OBS * np.log(np.float32(2.0 * np.pi)))
_NEG_LOG_N = np.float32(-np.log(np.float32(N)))


def _exact_inds(cum, u):
    b, n = cum.shape
    c = cum / cum[..., -1:]
    nf = jnp.float32(n)

    def p(i):
        return (u + i.astype(jnp.float32)) / nf

    t = jnp.clip((nf * c - u).astype(jnp.int32) + 1, 0, n)
    for _ in range(4):
        t = jnp.where((t > 0) & (p(t - 1) > c), t - 1, t)
    for _ in range(4):
        t = jnp.where((t < n) & (p(t) <= c), t + 1, t)
    hist = jnp.zeros((b, n), jnp.int32).at[
        jnp.arange(b)[:, None], jnp.clip(t, 0, n - 1)
    ].add(jnp.where(t <= n - 1, 1, 0))
    return jnp.clip(jnp.cumsum(hist, axis=-1), 0, n - 1)


def _tc_body(xT_ref, nzT_ref, lw_ref, mask_ref, y_ref, A_ref, C_ref,
             out_ref, mean_ref, ll_ref):
    xT = xT_ref[0]
    nzT = nzT_ref[0]
    A = A_ref[...]
    C = C_ref[...]
    y = y_ref[...].reshape(D_OBS, 1)

    x_new = jax.lax.dot(A, xT, preferred_element_type=jnp.float32) + 0.1 * nzT
    cx = jax.lax.dot(C, x_new, preferred_element_type=jnp.float32)
    q = (y - cx) * jnp.float32(2.0)
    logp = -0.5 * jnp.sum(q * q, axis=0, keepdims=True) + _LOGP_CONST

    lw_sel = jnp.where(mask_ref[0, 0, 0] > 0, _NEG_LOG_N, lw_ref[0])
    lw_new = lw_sel + logp
    out_ref[...] = lw_new[None]

    m = jnp.max(lw_new)
    e = jnp.exp(lw_new - m)
    s = jnp.sum(e)
    mean_ref[...] = (jnp.sum(x_new * e, axis=1) / s).reshape(1, 1, D)
    ll_ref[...] = (jnp.log(s) + m).reshape(1, 1, 1)


@functools.partial(jax.jit, static_argnames=())
def _tc_dense(x_selT, noiseT, logw_norm, maskf, y, A, C):
    out_shapes = (
        jax.ShapeDtypeStruct((B, 1, N), jnp.float32),
        jax.ShapeDtypeStruct((B, 1, D), jnp.float32),
        jax.ShapeDtypeStruct((B, 1, 1), jnp.float32),
    )
    return pl.pallas_call(
        _tc_body,
        grid=(B,),
        in_specs=[
            pl.BlockSpec((1, D, N), lambda b: (b, 0, 0)),
            pl.BlockSpec((1, D, N), lambda b: (b, 0, 0)),
            pl.BlockSpec((1, 1, N), lambda b: (b, 0, 0)),
            pl.BlockSpec((1, 1, 1), lambda b: (b, 0, 0)),
            pl.BlockSpec((1, 1, D_OBS), lambda b: (b, 0, 0)),
            pl.BlockSpec((D, D), lambda b: (0, 0)),
            pl.BlockSpec((D_OBS, D), lambda b: (0, 0)),
        ],
        out_specs=(
            pl.BlockSpec((1, 1, N), lambda b: (b, 0, 0)),
            pl.BlockSpec((1, 1, D), lambda b: (b, 0, 0)),
            pl.BlockSpec((1, 1, 1), lambda b: (b, 0, 0)),
        ),
        out_shape=out_shapes,
        compiler_params=pltpu.CompilerParams(
            dimension_semantics=("arbitrary",)),
    )(x_selT, noiseT, logw_norm, maskf, y, A, C)


def kernel(x_cur, log_w, y, noise, u, A, C):
    n = N
    logw_norm = log_w - jax.scipy.special.logsumexp(log_w, axis=-1, keepdims=True)
    w = jnp.exp(logw_norm)
    ess = 1.0 / jnp.sum(w * w, axis=-1) / n
    mask = ess < 0.9
    cum = jnp.cumsum(w, axis=-1)

    inds = _exact_inds(cum, u)
    x_res = jnp.take_along_axis(x_cur, inds[..., None], axis=1)
    x_sel = jnp.where(mask[:, None, None], x_res, x_cur)

    x_selT = jnp.transpose(x_sel, (0, 2, 1))
    noiseT = jnp.transpose(noise, (0, 2, 1))
    maskf = mask[:, None].astype(jnp.float32)

    log_w_new, mean, ll2 = _tc_dense(
        x_selT, noiseT, logw_norm[:, None, :], maskf[:, :, None],
        y[:, None, :], A, C)
    return (mean.reshape(B, D), ll2.reshape(B), log_w_new.reshape(B, N))


# speedup vs baseline: 5.1637x; 1.4844x over previous
import functools

import jax
import jax.numpy as jnp
import numpy as np
from jax import lax
from jax.experimental import pallas as pl
from jax.experimental.pallas import tpu as pltpu
from jax.experimental.pallas import tpu_sc as plsc

B, N, D, D_OBS = 16, 100000, 8, 4

_LOGP_CONST = np.float32(-D_OBS * np.log(np.float32(0.5))
                         - 0.5 * D_OBS * np.log(np.float32(2.0 * np.pi)))
_NEG_LOG_N = np.float32(-np.log(np.float32(N)))

_NW = 32
_H = N // 2
_K = 2500
_NR = _H // _K

_R16 = N // 16

_K_S = np.kron(np.eye(16, dtype=np.float32),
               np.ones((D_OBS, 1), np.float32))
_K_E = np.kron(np.eye(16, dtype=np.float32),
               np.ones((1, D), np.float32))
_K_F = np.kron(np.ones((16, 1), np.float32),
               np.eye(D, dtype=np.float32))


def _exact_inds(cum, u):
    b, n = cum.shape
    c = cum / cum[..., -1:]
    nf = jnp.float32(n)

    def p(i):
        return (u + i.astype(jnp.float32)) / nf

    t = jnp.clip((nf * c - u).astype(jnp.int32) + 1, 0, n)
    for _ in range(4):
        t = jnp.where((t > 0) & (p(t - 1) > c), t - 1, t)
    for _ in range(4):
        t = jnp.where((t < n) & (p(t) <= c), t + 1, t)
    hist = jnp.zeros((b, n), jnp.int32).at[
        jnp.arange(b)[:, None], jnp.clip(t, 0, n - 1)
    ].add(jnp.where(t <= n - 1, 1, 0))
    return jnp.clip(jnp.cumsum(hist, axis=-1), 0, n - 1)


def _sc_gather_body(xflat_hbm, idx_hbm, out_hbm, idx_v, rows_v, sem):
    wid = lax.axis_index("c") * 16 + lax.axis_index("s")

    def round_body(r, carry):
        slot = wid * _NR + r
        pltpu.sync_copy(idx_hbm.at[slot], idx_v)
        pltpu.async_copy(xflat_hbm.at[idx_v], rows_v, sem).wait()
        pltpu.sync_copy(rows_v, out_hbm.at[slot])
        return carry

    lax.fori_loop(0, _NR, round_body, 0)


@functools.partial(
    pl.kernel,
    mesh=plsc.VectorSubcoreMesh(core_axis_name="c", subcore_axis_name="s"),
    out_type=jax.ShapeDtypeStruct((_NW * _NR, _K, D), jnp.float32),
    compiler_params=pltpu.CompilerParams(use_tc_tiling_on_sc=False),
    scratch_types=[
        pltpu.VMEM((_K,), jnp.int32),
        pltpu.VMEM((_K, D), jnp.float32),
        pltpu.SemaphoreType.DMA,
    ],
)
def _sc_gather(xflat_hbm, idx_hbm, out_hbm, idx_v, rows_v, sem):
    _sc_gather_body(xflat_hbm, idx_hbm, out_hbm, idx_v, rows_v, sem)


def _tc_body(xi_ref, zi_ref, lw_ref, mask_ref, yt_ref, ka_ref, kc_ref,
             ks_ref, ke_ref, kf_ref, out_ref, mean_ref, ll_ref):
    xi = xi_ref[0]
    zi = zi_ref[0]
    ka = ka_ref[...]
    kc = kc_ref[...]
    yt = yt_ref[0]

    x_new = jax.lax.dot(xi, ka, preferred_element_type=jnp.float32) + 0.1 * zi
    cx = jax.lax.dot(x_new, kc, preferred_element_type=jnp.float32)
    q = (yt - cx) * jnp.float32(2.0)
    logp16 = jax.lax.dot(q * q, ks_ref[...],
                         preferred_element_type=jnp.float32)
    logp = jnp.float32(-0.5) * logp16 + _LOGP_CONST

    lw_sel = jnp.where(mask_ref[0, 0, 0] > 0, _NEG_LOG_N, lw_ref[0])
    lw_new = lw_sel + logp
    out_ref[0] = lw_new

    m = jnp.max(lw_new)
    e = jnp.exp(lw_new - m)
    s = jnp.sum(e)
    e_ib = jax.lax.dot(e, ke_ref[...],
                       preferred_element_type=jnp.float32)
    acc = jnp.sum(x_new * e_ib, axis=0).reshape(1, 128)
    mean = jax.lax.dot(acc, kf_ref[...],
                       preferred_element_type=jnp.float32) / s
    mean_ref[...] = mean.reshape(1, 1, D)
    ll_ref[...] = (jnp.log(s) + m).reshape(1, 1, 1)


@jax.jit
def _tc_dense(xi, zi, lw16, maskf, ytile, ka, kc, ks, ke, kf):
    out_shapes = (
        jax.ShapeDtypeStruct((B, _R16, 16), jnp.float32),
        jax.ShapeDtypeStruct((B, 1, D), jnp.float32),
        jax.ShapeDtypeStruct((B, 1, 1), jnp.float32),
    )
    return pl.pallas_call(
        _tc_body,
        grid=(B,),
        in_specs=[
            pl.BlockSpec((1, _R16, 128), lambda b: (b, 0, 0)),
            pl.BlockSpec((1, _R16, 128), lambda b: (b, 0, 0)),
            pl.BlockSpec((1, _R16, 16), lambda b: (b, 0, 0)),
            pl.BlockSpec((1, 1, 1), lambda b: (b, 0, 0)),
            pl.BlockSpec((1, 1, 4 * 16), lambda b: (b, 0, 0)),
            pl.BlockSpec((128, 128), lambda b: (0, 0)),
            pl.BlockSpec((128, 64), lambda b: (0, 0)),
            pl.BlockSpec((64, 16), lambda b: (0, 0)),
            pl.BlockSpec((16, 128), lambda b: (0, 0)),
            pl.BlockSpec((128, 8), lambda b: (0, 0)),
        ],
        out_specs=(
            pl.BlockSpec((1, _R16, 16), lambda b: (b, 0, 0)),
            pl.BlockSpec((1, 1, D), lambda b: (b, 0, 0)),
            pl.BlockSpec((1, 1, 1), lambda b: (b, 0, 0)),
        ),
        out_shape=out_shapes,
        compiler_params=pltpu.CompilerParams(
            dimension_semantics=("arbitrary",)),
    )(xi, zi, lw16, maskf, ytile, ka, kc, ks, ke, kf)


def kernel(x_cur, log_w, y, noise, u, A, C):
    n = N
    logw_norm = log_w - jax.scipy.special.logsumexp(log_w, axis=-1,
                                                    keepdims=True)
    w = jnp.exp(logw_norm)
    ess = 1.0 / jnp.sum(w * w, axis=-1) / n
    mask = ess < 0.9
    cum = jnp.cumsum(w, axis=-1)

    inds = _exact_inds(cum, u)
    iota = jnp.arange(n, dtype=jnp.int32)[None, :]
    inds_sel = jnp.where(mask[:, None], inds, iota)
    inds_off = inds_sel + (jnp.arange(B, dtype=jnp.int32) * N)[:, None]
    idx = inds_off.reshape(_NW * _NR, _K)
    xflat = x_cur.reshape(B * N, D)

    x_sel = _sc_gather(xflat, idx)

    xi = x_sel.reshape(B, _R16, 128)
    zi = noise.reshape(B, _R16, 128)
    lw16 = logw_norm.reshape(B, _R16, 16)
    maskf = mask[:, None, None].astype(jnp.float32)
    ytile = jnp.tile(y, (1, 16))[:, None, :]
    ka = jnp.kron(jnp.eye(16, dtype=jnp.float32), A.T)
    kc = jnp.kron(jnp.eye(16, dtype=jnp.float32), C.T)

    log_w_new, mean, ll = _tc_dense(
        xi, zi, lw16, maskf, ytile, ka, kc,
        jnp.asarray(_K_S), jnp.asarray(_K_E), jnp.asarray(_K_F))
    return (mean.reshape(B, D), ll.reshape(B), log_w_new.reshape(B, N))
